# baseline (speedup 1.0000x reference)
import functools

import jax
import jax.numpy as jnp
from jax import lax
from jax.experimental import pallas as pl
from jax.experimental.pallas import tpu as pltpu
from jax.experimental.pallas import tpu_sc as plsc

B, L, D, MAX_LEN = 4, 4096, 1024, 8192

NC, NS = 2, 16
NW = NC * NS
K = 2
LC = L // K
RPW = LC // NW
GCH = 32
NGC = RPW // GCH

_sc_mesh = plsc.VectorSubcoreMesh(core_axis_name="c", subcore_axis_name="s")


@functools.partial(
    pl.kernel,
    out_type=jax.ShapeDtypeStruct((LC, D), jnp.float32),
    mesh=_sc_mesh,
    scratch_types=[
        pltpu.VMEM((max(NGC, 2), GCH), jnp.int32),
        pltpu.VMEM((2, GCH, D), jnp.float32),
        pltpu.SemaphoreType.DMA,
        pltpu.SemaphoreType.DMA,
        pltpu.SemaphoreType.DMA,
    ],
)
def _sc_gather(pe_hbm, idx_hbm, out_hbm, idx_v, rows_v, sem_g0, sem_g1, sem_s):
    sem_g = (sem_g0, sem_g1)
    wid = lax.axis_index("s") * NC + lax.axis_index("c")
    base = wid * RPW
    pltpu.sync_copy(idx_hbm.at[wid], idx_v)
    gathers = [None] * NGC
    scatters = [None] * NGC
    gathers[0] = pltpu.async_copy(pe_hbm.at[idx_v.at[0]], rows_v.at[0], sem_g[0])
    for c in range(NGC):
        sl = c % 2
        if c + 1 < NGC:
            if c >= 1:
                scatters[c - 1].wait()
            gathers[c + 1] = pltpu.async_copy(
                pe_hbm.at[idx_v.at[c + 1]], rows_v.at[(c + 1) % 2],
                sem_g[(c + 1) % 2])
        gathers[c].wait()
        scatters[c] = pltpu.async_copy(
            rows_v.at[sl], out_hbm.at[pl.ds(base + c * GCH, GCH)], sem_s)
    if NGC >= 2:
        scatters[NGC - 2].wait()
    scatters[NGC - 1].wait()


_LB = 256


def _add_body(in_ref, g_ref, out_ref):
    out_ref[...] = in_ref[...] + g_ref[...][None, :, :]


def _add_body_carry(in_ref, g_ref, carry_ref, out_ref):
    out_ref[...] = in_ref[...] + g_ref[...][None, :, :]


def _make_tc_chunk(j):
    jofs = j * (LC // _LB)
    in_specs = [
        pl.BlockSpec((B, _LB, D), lambda i: (0, jofs + i, 0)),
        pl.BlockSpec((_LB, D), lambda i: (i, 0)),
    ]
    if j > 0:
        in_specs.append(pl.BlockSpec(memory_space=pl.ANY))
    return pl.pallas_call(
        _add_body_carry if j > 0 else _add_body,
        grid=(LC // _LB,),
        in_specs=in_specs,
        out_specs=pl.BlockSpec((B, _LB, D), lambda i: (0, jofs + i, 0)),
        out_shape=jax.ShapeDtypeStruct((B, L, D), jnp.float32),
        input_output_aliases={2: 0} if j > 0 else {},
        compiler_params=pltpu.CompilerParams(
            dimension_semantics=("arbitrary",),
        ),
    )


_tc_chunks = [_make_tc_chunk(j) for j in range(K)]


def kernel(input, indices, pe):
    idx = indices.astype(jnp.int32)
    gs = [
        _sc_gather(pe, idx[j * LC:(j + 1) * LC].reshape(NW, NGC, GCH))
        for j in range(K)
    ]
    out = _tc_chunks[0](input, gs[0])
    for j in range(1, K):
        out = _tc_chunks[j](input, gs[j], out)
    return out
